# Initial kernel scaffold; baseline (speedup 1.0000x reference)
import jax
import jax.numpy as jnp
from jax.experimental import pallas as pl


def kernel(h_node, edge_index, h_edge, ew1, eb1, ew2, eb2, eln_s, eln_b, nw1, nb1, nw2, nb2, nln_s, nln_b):
    raise NotImplementedError("write your pallas kernel here")


# speedup vs baseline: 3.7476x; 3.7476x over previous
import functools

import jax
import jax.numpy as jnp
from jax import lax
from jax.experimental import pallas as pl
from jax.experimental.pallas import tpu as pltpu
from jax.experimental.pallas import tpu_sc as plsc

N_NODES = 10000
N_EDGES = 320000
D = 128

SC_CORES = 2
SC_SUBCORES = 16
NW = SC_CORES * SC_SUBCORES
EPW = N_EDGES // NW
CHUNK = 400
NCHUNK = EPW // CHUNK
SCHUNK = 200
NSCHUNK = EPW // SCHUNK


def _f32_dot(x, w):
    return jax.lax.dot_general(x, w, (((1,), (0,)), ((), ())),
                               preferred_element_type=jnp.float32)


def _precompute_body(hn, a, b, pa, pb):
    x = hn[...]
    pa[...] = _f32_dot(x, a[...])
    pb[...] = _f32_dot(x, b[...])


def _precompute(h_node, a, b):
    R = 2000
    grid = (N_NODES // R,)
    return pl.pallas_call(
        _precompute_body,
        grid=grid,
        in_specs=[
            pl.BlockSpec((R, D), lambda i: (i, 0)),
            pl.BlockSpec((D, D), lambda i: (0, 0)),
            pl.BlockSpec((D, D), lambda i: (0, 0)),
        ],
        out_specs=[
            pl.BlockSpec((R, D), lambda i: (i, 0)),
            pl.BlockSpec((R, D), lambda i: (i, 0)),
        ],
        out_shape=[
            jax.ShapeDtypeStruct((N_NODES, D), jnp.float32),
            jax.ShapeDtypeStruct((N_NODES, D), jnp.float32),
        ],
    )(h_node, a, b)


def _edge_mlp_body(gs, gd, he, cw, w2, b1, b2, lns, lnb, out):
    h = he[...]
    x = gs[...] + gd[...] + _f32_dot(h, cw[...]) + b1[...]
    x = jnp.maximum(x, 0.0)
    e = _f32_dot(x, w2[...]) + b2[...]
    mu = jnp.mean(e, axis=-1, keepdims=True)
    var = jnp.mean((e - mu) ** 2, axis=-1, keepdims=True)
    e = (e - mu) * jax.lax.rsqrt(var + 1e-5) * lns[...] + lnb[...]
    out[...] = h + e


def _edge_mlp(gs, gd, h_edge, cw, w2, b1, b2, lns, lnb):
    R = 2000
    grid = (N_EDGES // R,)
    row = lambda i: (i, 0)
    full = lambda i: (0, 0)
    return pl.pallas_call(
        _edge_mlp_body,
        grid=grid,
        in_specs=[
            pl.BlockSpec((R, D), row),
            pl.BlockSpec((R, D), row),
            pl.BlockSpec((R, D), row),
            pl.BlockSpec((D, D), full),
            pl.BlockSpec((D, D), full),
            pl.BlockSpec((1, D), full),
            pl.BlockSpec((1, D), full),
            pl.BlockSpec((1, D), full),
            pl.BlockSpec((1, D), full),
        ],
        out_specs=pl.BlockSpec((R, D), row),
        out_shape=jax.ShapeDtypeStruct((N_EDGES, D), jnp.float32),
    )(gs, gd, h_edge, cw, w2, b1, b2, lns, lnb)


def _node_mlp_body(hn, p0, p1, wa, wb, w2, b1, b2, lns, lnb, out):
    h = hn[...]
    agg = p0[...] + p1[...]
    x = _f32_dot(h, wa[...]) + _f32_dot(agg, wb[...]) + b1[...]
    x = jnp.maximum(x, 0.0)
    n = _f32_dot(x, w2[...]) + b2[...]
    mu = jnp.mean(n, axis=-1, keepdims=True)
    var = jnp.mean((n - mu) ** 2, axis=-1, keepdims=True)
    n = (n - mu) * jax.lax.rsqrt(var + 1e-5) * lns[...] + lnb[...]
    out[...] = h + n


def _node_mlp(h_node, parts, wa, wb, w2, b1, b2, lns, lnb):
    R = 2000
    nb = N_NODES // R
    grid = (nb,)
    row = lambda i: (i, 0)
    full = lambda i: (0, 0)
    return pl.pallas_call(
        _node_mlp_body,
        grid=grid,
        in_specs=[
            pl.BlockSpec((R, D), row),
            pl.BlockSpec((R, D), row),
            pl.BlockSpec((R, D), lambda i, _nb=nb: (i + _nb, 0)),
            pl.BlockSpec((D, D), full),
            pl.BlockSpec((D, D), full),
            pl.BlockSpec((D, D), full),
            pl.BlockSpec((1, D), full),
            pl.BlockSpec((1, D), full),
            pl.BlockSpec((1, D), full),
            pl.BlockSpec((1, D), full),
        ],
        out_specs=pl.BlockSpec((R, D), row),
        out_shape=jax.ShapeDtypeStruct((N_NODES, D), jnp.float32),
    )(h_node, parts, parts, wa, wb, w2, b1, b2, lns, lnb)


def _sc_mesh():
    return plsc.VectorSubcoreMesh(
        core_axis_name="c", subcore_axis_name="s",
        num_cores=SC_CORES, num_subcores=SC_SUBCORES)


def _sc_gather(pa, pb, src, dst):
    @functools.partial(
        pl.kernel,
        out_type=[
            jax.ShapeDtypeStruct((N_EDGES, D), jnp.float32),
            jax.ShapeDtypeStruct((N_EDGES, D), jnp.float32),
        ],
        mesh=_sc_mesh(),
        scratch_types=[
            pltpu.VMEM((CHUNK,), jnp.int32),
            pltpu.VMEM((CHUNK,), jnp.int32),
            pltpu.VMEM((CHUNK, D), jnp.float32),
            pltpu.VMEM((CHUNK, D), jnp.float32),
            pltpu.SemaphoreType.DMA,
            pltpu.SemaphoreType.DMA,
        ],
    )
    def k(pa_hbm, pb_hbm, src_hbm, dst_hbm, gs_hbm, gd_hbm,
          idx_s, idx_d, rows_s, rows_d, sem_s, sem_d):
        wid = lax.axis_index("s") * SC_CORES + lax.axis_index("c")
        base = wid * EPW

        def body(kk, _):
            off = pl.multiple_of(base + kk * CHUNK, 8)
            pltpu.sync_copy(src_hbm.at[pl.ds(off, CHUNK)], idx_s)
            pltpu.sync_copy(dst_hbm.at[pl.ds(off, CHUNK)], idx_d)
            cp_s = pltpu.async_copy(pa_hbm.at[idx_s], rows_s, sem_s)
            cp_d = pltpu.async_copy(pb_hbm.at[idx_d], rows_d, sem_d)
            cp_s.wait()
            cp_d.wait()
            pltpu.sync_copy(rows_s, gs_hbm.at[pl.ds(off, CHUNK)])
            pltpu.sync_copy(rows_d, gd_hbm.at[pl.ds(off, CHUNK)])
            return ()

        lax.fori_loop(0, NCHUNK, body, ())

    return k(pa, pb, src, dst)


def _sc_segsum(h_edge, dst, zeros_tbl):
    @functools.partial(
        pl.kernel,
        out_type=jax.ShapeDtypeStruct((2 * N_NODES, D), jnp.float32),
        mesh=_sc_mesh(),
        scratch_types=[
            pltpu.VMEM((SCHUNK,), jnp.int32),
            pltpu.VMEM((SCHUNK, D), jnp.float32),
            pltpu.VMEM_SHARED((N_NODES, D), jnp.float32),
        ],
    )
    def k(he_hbm, dst_hbm, zeros_hbm, out_hbm, idx_v, rows_v, shared):
        c = lax.axis_index("c")
        s = lax.axis_index("s")
        base = (c * SC_SUBCORES + s) * EPW

        @pl.when(s == 0)
        def _zero():
            pltpu.sync_copy(zeros_hbm, shared)

        plsc.subcore_barrier()

        def body(kk, _):
            off = pl.multiple_of(base + kk * SCHUNK, 8)
            pltpu.sync_copy(dst_hbm.at[pl.ds(off, SCHUNK)], idx_v)
            pltpu.sync_copy(he_hbm.at[pl.ds(off, SCHUNK)], rows_v)
            pltpu.sync_copy(rows_v, shared.at[idx_v], add=True)
            return ()

        lax.fori_loop(0, NSCHUNK, body, ())

        plsc.subcore_barrier()

        rpt = 624
        r0 = s * rpt
        pltpu.sync_copy(shared.at[pl.ds(r0, rpt)],
                        out_hbm.at[pl.ds(c * N_NODES + r0, rpt)])

        @pl.when(s == 0)
        def _tail():
            t0 = SC_SUBCORES * rpt
            pltpu.sync_copy(shared.at[pl.ds(t0, N_NODES - t0)],
                            out_hbm.at[pl.ds(c * N_NODES + t0, N_NODES - t0)])

    return k(h_edge, dst, zeros_tbl)


def kernel(h_node, edge_index, h_edge, ew1, eb1, ew2, eb2, eln_s, eln_b,
           nw1, nb1, nw2, nb2, nln_s, nln_b):
    num_convs = ew1.shape[0]
    src = edge_index[0].astype(jnp.int32)
    dst = edge_index[1].astype(jnp.int32)
    zeros_tbl = jnp.zeros((N_NODES, D), jnp.float32)

    r1 = lambda v: v.reshape(1, D)

    for i in range(num_convs):
        a = ew1[i, :D]
        b = ew1[i, D:2 * D]
        cw = ew1[i, 2 * D:]
        pa, pb = _precompute(h_node, a, b)
        gs, gd = _sc_gather(pa, pb, src, dst)
        h_edge = _edge_mlp(gs, gd, h_edge, cw, ew2[i],
                           r1(eb1[i]), r1(eb2[i]), r1(eln_s[i]), r1(eln_b[i]))
        parts = _sc_segsum(h_edge, dst, zeros_tbl)
        h_node = _node_mlp(h_node, parts, nw1[i, :D], nw1[i, D:], nw2[i],
                           r1(nb1[i]), r1(nb2[i]), r1(nln_s[i]), r1(nln_b[i]))
    return (h_node, h_edge)


# speedup vs baseline: 3.9571x; 1.0559x over previous
import functools

import jax
import jax.numpy as jnp
from jax import lax
from jax.experimental import pallas as pl
from jax.experimental.pallas import tpu as pltpu
from jax.experimental.pallas import tpu_sc as plsc

N_NODES = 10000
N_EDGES = 320000
D = 128

SC_CORES = 2
SC_SUBCORES = 16
NW = SC_CORES * SC_SUBCORES
EPW = N_EDGES // NW
SCHUNK = 80
NSCHUNK = EPW // SCHUNK


def _f32_dot(x, w):
    return jax.lax.dot_general(x, w, (((1,), (0,)), ((), ())),
                               preferred_element_type=jnp.float32)


def _precompute_body(hn, a, b, pa, pb):
    x = hn[...]
    pa[...] = _f32_dot(x, a[...])
    pb[...] = _f32_dot(x, b[...])


def _precompute(h_node, a, b):
    R = 2000
    grid = (N_NODES // R,)
    return pl.pallas_call(
        _precompute_body,
        grid=grid,
        in_specs=[
            pl.BlockSpec((R, D), lambda i: (i, 0)),
            pl.BlockSpec((D, D), lambda i: (0, 0)),
            pl.BlockSpec((D, D), lambda i: (0, 0)),
        ],
        out_specs=[
            pl.BlockSpec((R, D), lambda i: (i, 0)),
            pl.BlockSpec((R, D), lambda i: (i, 0)),
        ],
        out_shape=[
            jax.ShapeDtypeStruct((N_NODES, D), jnp.float32),
            jax.ShapeDtypeStruct((N_NODES, D), jnp.float32),
        ],
    )(h_node, a, b)


def _edge_mlp_body(gs, gd, he, cw, w2, b1, b2, lns, lnb, out):
    h = he[...]
    x = gs[...] + gd[...] + _f32_dot(h, cw[...]) + b1[...]
    x = jnp.maximum(x, 0.0)
    e = _f32_dot(x, w2[...]) + b2[...]
    mu = jnp.mean(e, axis=-1, keepdims=True)
    var = jnp.mean((e - mu) ** 2, axis=-1, keepdims=True)
    e = (e - mu) * jax.lax.rsqrt(var + 1e-5) * lns[...] + lnb[...]
    out[...] = h + e


def _edge_mlp(gs, gd, h_edge, cw, w2, b1, b2, lns, lnb):
    R = 2000
    grid = (N_EDGES // R,)
    row = lambda i: (i, 0)
    full = lambda i: (0, 0)
    return pl.pallas_call(
        _edge_mlp_body,
        grid=grid,
        in_specs=[
            pl.BlockSpec((R, D), row),
            pl.BlockSpec((R, D), row),
            pl.BlockSpec((R, D), row),
            pl.BlockSpec((D, D), full),
            pl.BlockSpec((D, D), full),
            pl.BlockSpec((1, D), full),
            pl.BlockSpec((1, D), full),
            pl.BlockSpec((1, D), full),
            pl.BlockSpec((1, D), full),
        ],
        out_specs=pl.BlockSpec((R, D), row),
        out_shape=jax.ShapeDtypeStruct((N_EDGES, D), jnp.float32),
    )(gs, gd, h_edge, cw, w2, b1, b2, lns, lnb)


def _node_mlp_body(hn, p0, p1, wa, wb, w2, b1, b2, lns, lnb, out):
    h = hn[...]
    agg = p0[...] + p1[...]
    x = _f32_dot(h, wa[...]) + _f32_dot(agg, wb[...]) + b1[...]
    x = jnp.maximum(x, 0.0)
    n = _f32_dot(x, w2[...]) + b2[...]
    mu = jnp.mean(n, axis=-1, keepdims=True)
    var = jnp.mean((n - mu) ** 2, axis=-1, keepdims=True)
    n = (n - mu) * jax.lax.rsqrt(var + 1e-5) * lns[...] + lnb[...]
    out[...] = h + n


def _node_mlp(h_node, parts, wa, wb, w2, b1, b2, lns, lnb):
    R = 2000
    nb = N_NODES // R
    grid = (nb,)
    row = lambda i: (i, 0)
    full = lambda i: (0, 0)
    return pl.pallas_call(
        _node_mlp_body,
        grid=grid,
        in_specs=[
            pl.BlockSpec((R, D), row),
            pl.BlockSpec((R, D), row),
            pl.BlockSpec((R, D), lambda i, _nb=nb: (i + _nb, 0)),
            pl.BlockSpec((D, D), full),
            pl.BlockSpec((D, D), full),
            pl.BlockSpec((D, D), full),
            pl.BlockSpec((1, D), full),
            pl.BlockSpec((1, D), full),
            pl.BlockSpec((1, D), full),
            pl.BlockSpec((1, D), full),
        ],
        out_specs=pl.BlockSpec((R, D), row),
        out_shape=jax.ShapeDtypeStruct((N_NODES, D), jnp.float32),
    )(h_node, parts, parts, wa, wb, w2, b1, b2, lns, lnb)


def _sc_mesh():
    return plsc.VectorSubcoreMesh(
        core_axis_name="c", subcore_axis_name="s",
        num_cores=SC_CORES, num_subcores=SC_SUBCORES)


def _sc_gather(pa, pb, src, dst):
    C2 = 200
    NC2 = EPW // C2

    @functools.partial(
        pl.kernel,
        out_type=[
            jax.ShapeDtypeStruct((N_EDGES, D), jnp.float32),
            jax.ShapeDtypeStruct((N_EDGES, D), jnp.float32),
        ],
        mesh=_sc_mesh(),
        scratch_types=[
            pltpu.VMEM((EPW,), jnp.int32),
            pltpu.VMEM((EPW,), jnp.int32),
            pltpu.VMEM((C2, D), jnp.float32),
            pltpu.VMEM((C2, D), jnp.float32),
            pltpu.VMEM((C2, D), jnp.float32),
            pltpu.VMEM((C2, D), jnp.float32),
            pltpu.SemaphoreType.DMA,
            pltpu.SemaphoreType.DMA,
            pltpu.SemaphoreType.DMA,
            pltpu.SemaphoreType.DMA,
        ],
    )
    def k(pa_hbm, pb_hbm, src_hbm, dst_hbm, gs_hbm, gd_hbm,
          idx_s, idx_d, rs0, rd0, rs1, rd1, sg0, sg1, sw0, sw1):
        wid = lax.axis_index("s") * SC_CORES + lax.axis_index("c")
        base = wid * EPW
        pltpu.sync_copy(src_hbm.at[pl.ds(base, EPW)], idx_s)
        pltpu.sync_copy(dst_hbm.at[pl.ds(base, EPW)], idx_d)

        slots = ((rs0, rd0, sg0, sw0), (rs1, rd1, sg1, sw1))

        def g_copies(chunk, slot):
            rs, rd, sg, _ = slots[slot]
            o = pl.multiple_of(chunk * C2, 8)
            return (pltpu.make_async_copy(
                        pa_hbm.at[idx_s.at[pl.ds(o, C2)]], rs, sg),
                    pltpu.make_async_copy(
                        pb_hbm.at[idx_d.at[pl.ds(o, C2)]], rd, sg))

        def w_copies(chunk, slot):
            rs, rd, _, sw = slots[slot]
            o = pl.multiple_of(base + chunk * C2, 8)
            return (pltpu.make_async_copy(rs, gs_hbm.at[pl.ds(o, C2)], sw),
                    pltpu.make_async_copy(rd, gd_hbm.at[pl.ds(o, C2)], sw))

        def start_g(chunk, slot):
            for cp in g_copies(chunk, slot):
                cp.start()

        def wait_g(chunk, slot):
            for cp in g_copies(chunk, slot):
                cp.wait()

        def start_w(chunk, slot):
            for cp in w_copies(chunk, slot):
                cp.start()

        def wait_w(chunk, slot):
            for cp in w_copies(chunk, slot):
                cp.wait()

        start_g(0, 0)
        start_g(1, 1)

        def body(j, _):
            a = 2 * j
            for slot in (0, 1):
                c = a + slot
                wait_g(c, slot)
                start_w(c, slot)

            @pl.when(j < NC2 // 2 - 1)
            def _next():
                for slot in (0, 1):
                    c = a + slot
                    wait_w(c, slot)
                    start_g(c + 2, slot)

            return ()

        lax.fori_loop(0, NC2 // 2, body, ())
        wait_w(NC2 - 2, 0)
        wait_w(NC2 - 1, 1)

    return k(pa, pb, src, dst)


def _sc_segsum(h_edge, dst3, zeros_tbl):
    @functools.partial(
        pl.kernel,
        out_type=jax.ShapeDtypeStruct((2 * N_NODES, D), jnp.float32),
        mesh=_sc_mesh(),
        scratch_types=[
            pltpu.VMEM((NSCHUNK, SCHUNK), jnp.int32),
            pltpu.VMEM((SCHUNK, D), jnp.float32),
            pltpu.VMEM((SCHUNK, D), jnp.float32),
            pltpu.VMEM_SHARED((N_NODES, D), jnp.float32),
            pltpu.SemaphoreType.DMA,
            pltpu.SemaphoreType.DMA,
        ],
    )
    def k(he_hbm, dst_hbm, zeros_hbm, out_hbm, idx_v, r0, r1, shared,
          sl0, sl1):
        c = lax.axis_index("c")
        s = lax.axis_index("s")
        wid = c * SC_SUBCORES + s
        base = wid * EPW

        @pl.when(s == 0)
        def _zero():
            pltpu.sync_copy(zeros_hbm, shared)

        pltpu.sync_copy(dst_hbm.at[wid], idx_v)
        plsc.subcore_barrier()

        slots = ((r0, sl0), (r1, sl1))

        def load_copy(chunk, slot):
            rv, sl = slots[slot]
            off = pl.multiple_of(base + chunk * SCHUNK, 8)
            return pltpu.make_async_copy(he_hbm.at[pl.ds(off, SCHUNK)], rv, sl)

        def process(chunk, slot):
            rv, _ = slots[slot]
            load_copy(chunk, slot).wait()
            pltpu.sync_copy(rv, shared.at[idx_v.at[chunk]], add=True)

        load_copy(0, 0).start()
        load_copy(1, 1).start()

        def body(j, _):
            a = 2 * j
            for slot in (0, 1):
                process(a + slot, slot)

            @pl.when(j < NSCHUNK // 2 - 1)
            def _next():
                for slot in (0, 1):
                    load_copy(a + slot + 2, slot).start()

            return ()

        lax.fori_loop(0, NSCHUNK // 2, body, ())
        load_copy(NSCHUNK - 1, 0).start()
        process(NSCHUNK - 1, 0)

        plsc.subcore_barrier()

        rpt = 624
        r0 = s * rpt
        pltpu.sync_copy(shared.at[pl.ds(r0, rpt)],
                        out_hbm.at[pl.ds(c * N_NODES + r0, rpt)])

        @pl.when(s == 0)
        def _tail():
            t0 = SC_SUBCORES * rpt
            pltpu.sync_copy(shared.at[pl.ds(t0, N_NODES - t0)],
                            out_hbm.at[pl.ds(c * N_NODES + t0, N_NODES - t0)])

    return k(h_edge, dst3, zeros_tbl)


def kernel(h_node, edge_index, h_edge, ew1, eb1, ew2, eb2, eln_s, eln_b,
           nw1, nb1, nw2, nb2, nln_s, nln_b):
    num_convs = ew1.shape[0]
    src = edge_index[0].astype(jnp.int32)
    dst = edge_index[1].astype(jnp.int32)
    dst3 = dst.reshape(NW, NSCHUNK, SCHUNK)
    zeros_tbl = jnp.zeros((N_NODES, D), jnp.float32)

    r1 = lambda v: v.reshape(1, D)

    for i in range(num_convs):
        a = ew1[i, :D]
        b = ew1[i, D:2 * D]
        cw = ew1[i, 2 * D:]
        pa, pb = _precompute(h_node, a, b)
        gs, gd = _sc_gather(pa, pb, src, dst)
        h_edge = _edge_mlp(gs, gd, h_edge, cw, ew2[i],
                           r1(eb1[i]), r1(eb2[i]), r1(eln_s[i]), r1(eln_b[i]))
        parts = _sc_segsum(h_edge, dst3, zeros_tbl)
        h_node = _node_mlp(h_node, parts, nw1[i, :D], nw1[i, D:], nw2[i],
                           r1(nb1[i]), r1(nb2[i]), r1(nln_s[i]), r1(nln_b[i]))
    return (h_node, h_edge)


# speedup vs baseline: 4.1279x; 1.0432x over previous
import functools

import jax
import jax.numpy as jnp
from jax import lax
from jax.experimental import pallas as pl
from jax.experimental.pallas import tpu as pltpu
from jax.experimental.pallas import tpu_sc as plsc

N_NODES = 10000
N_EDGES = 320000
D = 128

SC_CORES = 2
SC_SUBCORES = 16
NW = SC_CORES * SC_SUBCORES
NSPLIT = 2
EG = N_EDGES // NSPLIT
EPW = EG // NW
GCHUNK = 200
NGCHUNK = EPW // GCHUNK
SCHUNK = 40
NSCHUNK = EPW // SCHUNK


def _f32_dot(x, w):
    return jax.lax.dot_general(x, w, (((1,), (0,)), ((), ())),
                               preferred_element_type=jnp.float32)


def _precompute_body(hn, a, b, pa, pb):
    x = hn[...]
    pa[...] = _f32_dot(x, a[...])
    pb[...] = _f32_dot(x, b[...])


def _precompute(h_node, a, b):
    R = 2000
    grid = (N_NODES // R,)
    return pl.pallas_call(
        _precompute_body,
        grid=grid,
        in_specs=[
            pl.BlockSpec((R, D), lambda i: (i, 0)),
            pl.BlockSpec((D, D), lambda i: (0, 0)),
            pl.BlockSpec((D, D), lambda i: (0, 0)),
        ],
        out_specs=[
            pl.BlockSpec((R, D), lambda i: (i, 0)),
            pl.BlockSpec((R, D), lambda i: (i, 0)),
        ],
        out_shape=[
            jax.ShapeDtypeStruct((N_NODES, D), jnp.float32),
            jax.ShapeDtypeStruct((N_NODES, D), jnp.float32),
        ],
    )(h_node, a, b)


def _edge_mlp_body(gs, gd, he, cw, w2, b1, b2, lns, lnb, out):
    h = he[...]
    x = gs[...] + gd[...] + _f32_dot(h, cw[...]) + b1[...]
    x = jnp.maximum(x, 0.0)
    e = _f32_dot(x, w2[...]) + b2[...]
    mu = jnp.mean(e, axis=-1, keepdims=True)
    var = jnp.mean((e - mu) ** 2, axis=-1, keepdims=True)
    e = (e - mu) * jax.lax.rsqrt(var + 1e-5) * lns[...] + lnb[...]
    out[...] = h + e


def _edge_mlp(gs, gd, h_edge, cw, w2, b1, b2, lns, lnb):
    R = 2000
    grid = (EG // R,)
    row = lambda i: (i, 0)
    full = lambda i: (0, 0)
    return pl.pallas_call(
        _edge_mlp_body,
        grid=grid,
        in_specs=[
            pl.BlockSpec((R, D), row),
            pl.BlockSpec((R, D), row),
            pl.BlockSpec((R, D), row),
            pl.BlockSpec((D, D), full),
            pl.BlockSpec((D, D), full),
            pl.BlockSpec((1, D), full),
            pl.BlockSpec((1, D), full),
            pl.BlockSpec((1, D), full),
            pl.BlockSpec((1, D), full),
        ],
        out_specs=pl.BlockSpec((R, D), row),
        out_shape=jax.ShapeDtypeStruct((EG, D), jnp.float32),
    )(gs, gd, h_edge, cw, w2, b1, b2, lns, lnb)


def _node_mlp_body(hn, p0, p1, wa, wb, w2, b1, b2, lns, lnb, out):
    h = hn[...]
    agg = p0[...] + p1[...]
    x = _f32_dot(h, wa[...]) + _f32_dot(agg, wb[...]) + b1[...]
    x = jnp.maximum(x, 0.0)
    n = _f32_dot(x, w2[...]) + b2[...]
    mu = jnp.mean(n, axis=-1, keepdims=True)
    var = jnp.mean((n - mu) ** 2, axis=-1, keepdims=True)
    n = (n - mu) * jax.lax.rsqrt(var + 1e-5) * lns[...] + lnb[...]
    out[...] = h + n


def _node_mlp(h_node, parts, wa, wb, w2, b1, b2, lns, lnb):
    R = 2000
    nb = N_NODES // R
    grid = (nb,)
    row = lambda i: (i, 0)
    full = lambda i: (0, 0)
    return pl.pallas_call(
        _node_mlp_body,
        grid=grid,
        in_specs=[
            pl.BlockSpec((R, D), row),
            pl.BlockSpec((R, D), row),
            pl.BlockSpec((R, D), lambda i, _nb=nb: (i + _nb, 0)),
            pl.BlockSpec((D, D), full),
            pl.BlockSpec((D, D), full),
            pl.BlockSpec((D, D), full),
            pl.BlockSpec((1, D), full),
            pl.BlockSpec((1, D), full),
            pl.BlockSpec((1, D), full),
            pl.BlockSpec((1, D), full),
        ],
        out_specs=pl.BlockSpec((R, D), row),
        out_shape=jax.ShapeDtypeStruct((N_NODES, D), jnp.float32),
    )(h_node, parts, parts, wa, wb, w2, b1, b2, lns, lnb)


def _sc_mesh():
    return plsc.VectorSubcoreMesh(
        core_axis_name="c", subcore_axis_name="s",
        num_cores=SC_CORES, num_subcores=SC_SUBCORES)


def _sc_gather(pa, pb, src, dst):
    C2 = GCHUNK
    NC2 = NGCHUNK

    @functools.partial(
        pl.kernel,
        out_type=[
            jax.ShapeDtypeStruct((EG, D), jnp.float32),
            jax.ShapeDtypeStruct((EG, D), jnp.float32),
        ],
        mesh=_sc_mesh(),
        scratch_types=[
            pltpu.VMEM((EPW,), jnp.int32),
            pltpu.VMEM((EPW,), jnp.int32),
            pltpu.VMEM((C2, D), jnp.float32),
            pltpu.VMEM((C2, D), jnp.float32),
            pltpu.VMEM((C2, D), jnp.float32),
            pltpu.VMEM((C2, D), jnp.float32),
            pltpu.SemaphoreType.DMA,
            pltpu.SemaphoreType.DMA,
            pltpu.SemaphoreType.DMA,
            pltpu.SemaphoreType.DMA,
        ],
    )
    def k(pa_hbm, pb_hbm, src_hbm, dst_hbm, gs_hbm, gd_hbm,
          idx_s, idx_d, rs0, rd0, rs1, rd1, sg0, sg1, sw0, sw1):
        wid = lax.axis_index("s") * SC_CORES + lax.axis_index("c")
        base = wid * EPW
        pltpu.sync_copy(src_hbm.at[pl.ds(base, EPW)], idx_s)
        pltpu.sync_copy(dst_hbm.at[pl.ds(base, EPW)], idx_d)

        slots = ((rs0, rd0, sg0, sw0), (rs1, rd1, sg1, sw1))

        def g_copies(chunk, slot):
            rs, rd, sg, _ = slots[slot]
            o = pl.multiple_of(chunk * C2, 8)
            return (pltpu.make_async_copy(
                        pa_hbm.at[idx_s.at[pl.ds(o, C2)]], rs, sg),
                    pltpu.make_async_copy(
                        pb_hbm.at[idx_d.at[pl.ds(o, C2)]], rd, sg))

        def w_copies(chunk, slot):
            rs, rd, _, sw = slots[slot]
            o = pl.multiple_of(base + chunk * C2, 8)
            return (pltpu.make_async_copy(rs, gs_hbm.at[pl.ds(o, C2)], sw),
                    pltpu.make_async_copy(rd, gd_hbm.at[pl.ds(o, C2)], sw))

        def start_g(chunk, slot):
            for cp in g_copies(chunk, slot):
                cp.start()

        def wait_g(chunk, slot):
            for cp in g_copies(chunk, slot):
                cp.wait()

        def start_w(chunk, slot):
            for cp in w_copies(chunk, slot):
                cp.start()

        def wait_w(chunk, slot):
            for cp in w_copies(chunk, slot):
                cp.wait()

        start_g(0, 0)
        start_g(1, 1)

        npairs = NC2 // 2

        def body(j, _):
            a = 2 * j
            for slot in (0, 1):
                c = a + slot
                wait_g(c, slot)
                start_w(c, slot)

            @pl.when(j < npairs - 1)
            def _next():
                for slot in (0, 1):
                    c = a + slot
                    wait_w(c, slot)
                    start_g(c + 2, slot)

            return ()

        lax.fori_loop(0, npairs, body, ())
        if NC2 % 2:
            last = NC2 - 1
            wait_w(last - 2, 0)
            start_g(last, 0)
            wait_g(last, 0)
            start_w(last, 0)
            wait_w(last - 1, 1)
            wait_w(last, 0)
        else:
            wait_w(NC2 - 2, 0)
            wait_w(NC2 - 1, 1)

    return k(pa, pb, src, dst)


def _sc_segsum(h_edge, dst3, init):
    @functools.partial(
        pl.kernel,
        out_type=jax.ShapeDtypeStruct((2 * N_NODES, D), jnp.float32),
        mesh=_sc_mesh(),
        scratch_types=[
            pltpu.VMEM((NSCHUNK, SCHUNK), jnp.int32),
            pltpu.VMEM((SCHUNK, D), jnp.float32),
            pltpu.VMEM((SCHUNK, D), jnp.float32),
            pltpu.VMEM_SHARED((N_NODES, D), jnp.float32),
            pltpu.SemaphoreType.DMA,
            pltpu.SemaphoreType.DMA,
        ],
    )
    def k(he_hbm, dst_hbm, init_hbm, out_hbm, idx_v, r0, r1, shared,
          sl0, sl1):
        c = lax.axis_index("c")
        s = lax.axis_index("s")
        wid = c * SC_SUBCORES + s
        base = wid * EPW

        @pl.when(s == 0)
        def _load_init():
            pltpu.sync_copy(init_hbm.at[pl.ds(c * N_NODES, N_NODES)], shared)

        pltpu.sync_copy(dst_hbm.at[wid], idx_v)
        plsc.subcore_barrier()

        slots = ((r0, sl0), (r1, sl1))

        def load_copy(chunk, slot):
            rv, sl = slots[slot]
            off = pl.multiple_of(base + chunk * SCHUNK, 8)
            return pltpu.make_async_copy(he_hbm.at[pl.ds(off, SCHUNK)], rv, sl)

        def process(chunk, slot):
            rv, _ = slots[slot]
            load_copy(chunk, slot).wait()
            pltpu.sync_copy(rv, shared.at[idx_v.at[chunk]], add=True)

        load_copy(0, 0).start()
        load_copy(1, 1).start()

        def body(j, _):
            a = 2 * j
            for slot in (0, 1):
                process(a + slot, slot)

            @pl.when(j < NSCHUNK // 2 - 1)
            def _next():
                for slot in (0, 1):
                    load_copy(a + slot + 2, slot).start()

            return ()

        lax.fori_loop(0, NSCHUNK // 2, body, ())
        load_copy(NSCHUNK - 1, 0).start()
        process(NSCHUNK - 1, 0)

        plsc.subcore_barrier()

        rpt = 624
        r0 = s * rpt
        pltpu.sync_copy(shared.at[pl.ds(r0, rpt)],
                        out_hbm.at[pl.ds(c * N_NODES + r0, rpt)])

        @pl.when(s == 0)
        def _tail():
            t0 = SC_SUBCORES * rpt
            pltpu.sync_copy(shared.at[pl.ds(t0, N_NODES - t0)],
                            out_hbm.at[pl.ds(c * N_NODES + t0, N_NODES - t0)])

    return k(h_edge, dst3, init)


def kernel(h_node, edge_index, h_edge, ew1, eb1, ew2, eb2, eln_s, eln_b,
           nw1, nb1, nw2, nb2, nln_s, nln_b):
    num_convs = ew1.shape[0]
    src = edge_index[0].astype(jnp.int32)
    dst = edge_index[1].astype(jnp.int32)
    src_g = [src[g * EG:(g + 1) * EG] for g in range(NSPLIT)]
    dst_g = [dst[g * EG:(g + 1) * EG] for g in range(NSPLIT)]
    dst3_g = [d.reshape(NW, NSCHUNK, SCHUNK) for d in dst_g]
    he_g = [h_edge[g * EG:(g + 1) * EG] for g in range(NSPLIT)]
    zeros2 = jnp.zeros((2 * N_NODES, D), jnp.float32)

    r1 = lambda v: v.reshape(1, D)

    for i in range(num_convs):
        a = ew1[i, :D]
        b = ew1[i, D:2 * D]
        cw = ew1[i, 2 * D:]
        pa, pb = _precompute(h_node, a, b)
        gg = [_sc_gather(pa, pb, src_g[g], dst_g[g]) for g in range(NSPLIT)]
        he_g = [_edge_mlp(gg[g][0], gg[g][1], he_g[g], cw, ew2[i],
                          r1(eb1[i]), r1(eb2[i]), r1(eln_s[i]), r1(eln_b[i]))
                for g in range(NSPLIT)]
        parts = zeros2
        for g in range(NSPLIT):
            parts = _sc_segsum(he_g[g], dst3_g[g], parts)
        h_node = _node_mlp(h_node, parts, nw1[i, :D], nw1[i, D:], nw2[i],
                           r1(nb1[i]), r1(nb2[i]), r1(nln_s[i]), r1(nln_b[i]))
    return (h_node, jnp.concatenate(he_g, axis=0))


# speedup vs baseline: 4.7294x; 1.1457x over previous
import functools

import jax
import jax.numpy as jnp
from jax import lax
from jax.experimental import pallas as pl
from jax.experimental.pallas import tpu as pltpu
from jax.experimental.pallas import tpu_sc as plsc

N_NODES = 10000
N_EDGES = 320000
D = 128

SC_CORES = 2
SC_SUBCORES = 16
NW = SC_CORES * SC_SUBCORES
NSPLIT = 2
EG = N_EDGES // NSPLIT
EPW = EG // NW
GCHUNK = 200
NGCHUNK = EPW // GCHUNK
SCHUNK = 40
NSCHUNK = EPW // SCHUNK


def _f32_dot(x, w):
    return jax.lax.dot_general(x, w, (((1,), (0,)), ((), ())),
                               preferred_element_type=jnp.float32)


def _precompute_body(hn, a, b, pa, pb):
    x = hn[...]
    pa[...] = _f32_dot(x, a[...])
    pb[...] = _f32_dot(x, b[...])


def _precompute(h_node, a, b):
    R = 2000
    grid = (N_NODES // R,)
    return pl.pallas_call(
        _precompute_body,
        grid=grid,
        in_specs=[
            pl.BlockSpec((R, D), lambda i: (i, 0)),
            pl.BlockSpec((D, D), lambda i: (0, 0)),
            pl.BlockSpec((D, D), lambda i: (0, 0)),
        ],
        out_specs=[
            pl.BlockSpec((R, D), lambda i: (i, 0)),
            pl.BlockSpec((R, D), lambda i: (i, 0)),
        ],
        out_shape=[
            jax.ShapeDtypeStruct((N_NODES, D), jnp.float32),
            jax.ShapeDtypeStruct((N_NODES, D), jnp.float32),
        ],
    )(h_node, a, b)


def _edge_mlp_body(gsum, he, cw, w2, b1, b2, lns, lnb, out):
    h = he[...]
    x = gsum[...] + _f32_dot(h, cw[...]) + b1[...]
    x = jnp.maximum(x, 0.0)
    e = _f32_dot(x, w2[...]) + b2[...]
    mu = jnp.mean(e, axis=-1, keepdims=True)
    var = jnp.mean((e - mu) ** 2, axis=-1, keepdims=True)
    e = (e - mu) * jax.lax.rsqrt(var + 1e-5) * lns[...] + lnb[...]
    out[...] = h + e


def _edge_mlp(gsum, h_edge, cw, w2, b1, b2, lns, lnb):
    R = 2000
    grid = (EG // R,)
    row = lambda i: (i, 0)
    full = lambda i: (0, 0)
    return pl.pallas_call(
        _edge_mlp_body,
        grid=grid,
        in_specs=[
            pl.BlockSpec((R, D), row),
            pl.BlockSpec((R, D), row),
            pl.BlockSpec((D, D), full),
            pl.BlockSpec((D, D), full),
            pl.BlockSpec((1, D), full),
            pl.BlockSpec((1, D), full),
            pl.BlockSpec((1, D), full),
            pl.BlockSpec((1, D), full),
        ],
        out_specs=pl.BlockSpec((R, D), row),
        out_shape=jax.ShapeDtypeStruct((EG, D), jnp.float32),
    )(gsum, h_edge, cw, w2, b1, b2, lns, lnb)


def _node_mlp_body(hn, p0, p1, wa, wb, w2, b1, b2, lns, lnb, out):
    h = hn[...]
    agg = p0[...] + p1[...]
    x = _f32_dot(h, wa[...]) + _f32_dot(agg, wb[...]) + b1[...]
    x = jnp.maximum(x, 0.0)
    n = _f32_dot(x, w2[...]) + b2[...]
    mu = jnp.mean(n, axis=-1, keepdims=True)
    var = jnp.mean((n - mu) ** 2, axis=-1, keepdims=True)
    n = (n - mu) * jax.lax.rsqrt(var + 1e-5) * lns[...] + lnb[...]
    out[...] = h + n


def _node_mlp(h_node, parts, wa, wb, w2, b1, b2, lns, lnb):
    R = 2000
    nb = N_NODES // R
    grid = (nb,)
    row = lambda i: (i, 0)
    full = lambda i: (0, 0)
    return pl.pallas_call(
        _node_mlp_body,
        grid=grid,
        in_specs=[
            pl.BlockSpec((R, D), row),
            pl.BlockSpec((R, D), row),
            pl.BlockSpec((R, D), lambda i, _nb=nb: (i + _nb, 0)),
            pl.BlockSpec((D, D), full),
            pl.BlockSpec((D, D), full),
            pl.BlockSpec((D, D), full),
            pl.BlockSpec((1, D), full),
            pl.BlockSpec((1, D), full),
            pl.BlockSpec((1, D), full),
            pl.BlockSpec((1, D), full),
        ],
        out_specs=pl.BlockSpec((R, D), row),
        out_shape=jax.ShapeDtypeStruct((N_NODES, D), jnp.float32),
    )(h_node, parts, parts, wa, wb, w2, b1, b2, lns, lnb)


def _sc_mesh():
    return plsc.VectorSubcoreMesh(
        core_axis_name="c", subcore_axis_name="s",
        num_cores=SC_CORES, num_subcores=SC_SUBCORES)


def _sc_gather(pa, pb, src, dst):
    C2 = GCHUNK
    NC2 = NGCHUNK

    @functools.partial(
        pl.kernel,
        out_type=jax.ShapeDtypeStruct((EG, D), jnp.float32),
        mesh=_sc_mesh(),
        scratch_types=[
            pltpu.VMEM((EPW,), jnp.int32),
            pltpu.VMEM((EPW,), jnp.int32),
            pltpu.VMEM((C2, D), jnp.float32),
            pltpu.VMEM((C2, D), jnp.float32),
            pltpu.VMEM((C2, D), jnp.float32),
            pltpu.VMEM((C2, D), jnp.float32),
            pltpu.SemaphoreType.DMA,
            pltpu.SemaphoreType.DMA,
            pltpu.SemaphoreType.DMA,
            pltpu.SemaphoreType.DMA,
        ],
    )
    def k(pa_hbm, pb_hbm, src_hbm, dst_hbm, gsum_hbm,
          idx_s, idx_d, rs0, rd0, rs1, rd1, sg0, sg1, sw0, sw1):
        wid = lax.axis_index("s") * SC_CORES + lax.axis_index("c")
        base = wid * EPW
        pltpu.sync_copy(src_hbm.at[pl.ds(base, EPW)], idx_s)
        pltpu.sync_copy(dst_hbm.at[pl.ds(base, EPW)], idx_d)

        slots = ((rs0, rd0, sg0, sw0), (rs1, rd1, sg1, sw1))

        def g_copies(chunk, slot):
            rs, rd, sg, _ = slots[slot]
            o = pl.multiple_of(chunk * C2, 8)
            return (pltpu.make_async_copy(
                        pa_hbm.at[idx_s.at[pl.ds(o, C2)]], rs, sg),
                    pltpu.make_async_copy(
                        pb_hbm.at[idx_d.at[pl.ds(o, C2)]], rd, sg))

        def w_copy(chunk, slot):
            rs, _, _, sw = slots[slot]
            o = pl.multiple_of(base + chunk * C2, 8)
            return pltpu.make_async_copy(rs, gsum_hbm.at[pl.ds(o, C2)], sw)

        def start_g(chunk, slot):
            for cp in g_copies(chunk, slot):
                cp.start()

        def wait_g(chunk, slot):
            for cp in g_copies(chunk, slot):
                cp.wait()

        def vsum(slot):
            rs, rd, _, _ = slots[slot]

            @plsc.parallel_loop(0, C2, 1, unroll=2)
            def _add(r):
                for cc in range(D // 16):
                    sl = pl.ds(cc * 16, 16)
                    rs[r, sl] = rs[r, sl] + rd[r, sl]

        start_g(0, 0)
        start_g(1, 1)

        npairs = NC2 // 2

        def body(j, _):
            a = 2 * j
            for slot in (0, 1):
                c = a + slot
                wait_g(c, slot)
                vsum(slot)
                w_copy(c, slot).start()

            @pl.when(j < npairs - 1)
            def _next():
                for slot in (0, 1):
                    c = a + slot
                    w_copy(c, slot).wait()
                    start_g(c + 2, slot)

            return ()

        lax.fori_loop(0, npairs, body, ())
        if NC2 % 2:
            last = NC2 - 1
            w_copy(last - 2, 0).wait()
            start_g(last, 0)
            wait_g(last, 0)
            vsum(0)
            w_copy(last, 0).start()
            w_copy(last - 1, 1).wait()
            w_copy(last, 0).wait()
        else:
            w_copy(NC2 - 2, 0).wait()
            w_copy(NC2 - 1, 1).wait()

    return k(pa, pb, src, dst)


def _sc_segsum(h_edge, dst3, init):
    @functools.partial(
        pl.kernel,
        out_type=jax.ShapeDtypeStruct((2 * N_NODES, D), jnp.float32),
        mesh=_sc_mesh(),
        scratch_types=[
            pltpu.VMEM((NSCHUNK, SCHUNK), jnp.int32),
            pltpu.VMEM((SCHUNK, D), jnp.float32),
            pltpu.VMEM((SCHUNK, D), jnp.float32),
            pltpu.VMEM_SHARED((N_NODES, D), jnp.float32),
            pltpu.SemaphoreType.DMA,
            pltpu.SemaphoreType.DMA,
        ],
    )
    def k(he_hbm, dst_hbm, init_hbm, out_hbm, idx_v, r0, r1, shared,
          sl0, sl1):
        c = lax.axis_index("c")
        s = lax.axis_index("s")
        wid = c * SC_SUBCORES + s
        base = wid * EPW

        @pl.when(s == 0)
        def _load_init():
            pltpu.sync_copy(init_hbm.at[pl.ds(c * N_NODES, N_NODES)], shared)

        pltpu.sync_copy(dst_hbm.at[wid], idx_v)
        plsc.subcore_barrier()

        slots = ((r0, sl0), (r1, sl1))

        def load_copy(chunk, slot):
            rv, sl = slots[slot]
            off = pl.multiple_of(base + chunk * SCHUNK, 8)
            return pltpu.make_async_copy(he_hbm.at[pl.ds(off, SCHUNK)], rv, sl)

        def process(chunk, slot):
            rv, _ = slots[slot]
            load_copy(chunk, slot).wait()
            pltpu.sync_copy(rv, shared.at[idx_v.at[chunk]], add=True)

        load_copy(0, 0).start()
        load_copy(1, 1).start()

        def body(j, _):
            a = 2 * j
            for slot in (0, 1):
                process(a + slot, slot)

            @pl.when(j < NSCHUNK // 2 - 1)
            def _next():
                for slot in (0, 1):
                    load_copy(a + slot + 2, slot).start()

            return ()

        lax.fori_loop(0, NSCHUNK // 2, body, ())
        load_copy(NSCHUNK - 1, 0).start()
        process(NSCHUNK - 1, 0)

        plsc.subcore_barrier()

        rpt = 624
        r0 = s * rpt
        pltpu.sync_copy(shared.at[pl.ds(r0, rpt)],
                        out_hbm.at[pl.ds(c * N_NODES + r0, rpt)])

        @pl.when(s == 0)
        def _tail():
            t0 = SC_SUBCORES * rpt
            pltpu.sync_copy(shared.at[pl.ds(t0, N_NODES - t0)],
                            out_hbm.at[pl.ds(c * N_NODES + t0, N_NODES - t0)])

    return k(h_edge, dst3, init)


def kernel(h_node, edge_index, h_edge, ew1, eb1, ew2, eb2, eln_s, eln_b,
           nw1, nb1, nw2, nb2, nln_s, nln_b):
    num_convs = ew1.shape[0]
    src = edge_index[0].astype(jnp.int32)
    dst = edge_index[1].astype(jnp.int32)
    src_g = [src[g * EG:(g + 1) * EG] for g in range(NSPLIT)]
    dst_g = [dst[g * EG:(g + 1) * EG] for g in range(NSPLIT)]
    dst3_g = [d.reshape(NW, NSCHUNK, SCHUNK) for d in dst_g]
    he_g = [h_edge[g * EG:(g + 1) * EG] for g in range(NSPLIT)]
    zeros2 = jnp.zeros((2 * N_NODES, D), jnp.float32)

    r1 = lambda v: v.reshape(1, D)

    for i in range(num_convs):
        a = ew1[i, :D]
        b = ew1[i, D:2 * D]
        cw = ew1[i, 2 * D:]
        pa, pb = _precompute(h_node, a, b)
        gg = [_sc_gather(pa, pb, src_g[g], dst_g[g]) for g in range(NSPLIT)]
        he_g = [_edge_mlp(gg[g], he_g[g], cw, ew2[i],
                          r1(eb1[i]), r1(eb2[i]), r1(eln_s[i]), r1(eln_b[i]))
                for g in range(NSPLIT)]
        parts = zeros2
        for g in range(NSPLIT):
            parts = _sc_segsum(he_g[g], dst3_g[g], parts)
        h_node = _node_mlp(h_node, parts, nw1[i, :D], nw1[i, D:], nw2[i],
                           r1(nb1[i]), r1(nb2[i]), r1(nln_s[i]), r1(nln_b[i]))
    return (h_node, jnp.concatenate(he_g, axis=0))
